# Initial kernel scaffold; baseline (speedup 1.0000x reference)
import jax
import jax.numpy as jnp
from jax.experimental import pallas as pl


def kernel(inputs, emb, W_ih, W_hh, b_ih, b_hh, W_out, b_out):
    raise NotImplementedError("write your pallas kernel here")


# speedup vs baseline: 3.3352x; 3.3352x over previous
import functools

import jax
import jax.numpy as jnp
from jax.experimental import pallas as pl
from jax.experimental.pallas import tpu as pltpu

S_BLK = 32


def _rnn_body(x_ref, wih_ref, whh_ref, bih_ref, bhh_ref, wout_ref,
              bout_ref, out_ref, xw_ref, h_ref, acc_ref, *, ns, s_total):
    j = pl.program_id(1)
    bh = h_ref.shape[0]

    @pl.when(j == 0)
    def _():
        h_ref[...] = jnp.zeros_like(h_ref)
        acc_ref[...] = jnp.zeros_like(acc_ref)

    xb = x_ref[...]
    e_dim = xb.shape[-1]
    xw_ref[...] = (
        jnp.dot(xb.reshape(S_BLK * bh, e_dim), wih_ref[...],
                preferred_element_type=jnp.float32)
        + bih_ref[...]
    )

    h = h_ref[...]
    acc = acc_ref[...]
    whh = whh_ref[...]
    bhh = bhh_ref[...]
    for t in range(S_BLK):
        xw_t = xw_ref[t * bh:(t + 1) * bh, :]
        h = jnp.tanh(xw_t + jnp.dot(h, whh, preferred_element_type=jnp.float32)
                     + bhh)
        acc = acc + h
    h_ref[...] = h
    acc_ref[...] = acc

    @pl.when(j == ns - 1)
    def _():
        z = (jnp.dot(acc, wout_ref[...], preferred_element_type=jnp.float32)
             + s_total * bout_ref[...])
        m = jnp.max(z, axis=1, keepdims=True)
        lse = jnp.log(jnp.sum(jnp.exp(z - m), axis=1, keepdims=True)) + m
        out_ref[...] = z - lse


def kernel(inputs, emb, W_ih, W_hh, b_ih, b_hh, W_out, b_out):
    S, B = inputs.shape
    V, E = emb.shape
    H = W_hh.shape[0]
    C = W_out.shape[0]
    ns = S // S_BLK
    bh = B // 2

    x = emb[inputs.astype(jnp.int32)]

    body = functools.partial(_rnn_body, ns=ns, s_total=float(S))

    out = pl.pallas_call(
        body,
        out_shape=jax.ShapeDtypeStruct((B, C), jnp.float32),
        grid=(2, ns),
        in_specs=[
            pl.BlockSpec((S_BLK, bh, E), lambda i, j: (j, i, 0)),
            pl.BlockSpec((E, H), lambda i, j: (0, 0)),
            pl.BlockSpec((H, H), lambda i, j: (0, 0)),
            pl.BlockSpec((1, H), lambda i, j: (0, 0)),
            pl.BlockSpec((1, H), lambda i, j: (0, 0)),
            pl.BlockSpec((H, C), lambda i, j: (0, 0)),
            pl.BlockSpec((1, C), lambda i, j: (0, 0)),
        ],
        out_specs=pl.BlockSpec((bh, C), lambda i, j: (i, 0)),
        scratch_shapes=[
            pltpu.VMEM((S_BLK * bh, H), jnp.float32),
            pltpu.VMEM((bh, H), jnp.float32),
            pltpu.VMEM((bh, H), jnp.float32),
        ],
        compiler_params=pltpu.CompilerParams(
            dimension_semantics=("parallel", "arbitrary"),
        ),
        name="rnn_fused",
    )(
        x,
        W_ih.T,
        W_hh.T,
        b_ih.reshape(1, H),
        b_hh.reshape(1, H),
        W_out.T,
        b_out.reshape(1, C),
    )
    return out


# speedup vs baseline: 4.3414x; 1.3017x over previous
import functools

import jax
import jax.numpy as jnp
from jax.experimental import pallas as pl
from jax.experimental.pallas import tpu as pltpu

S_BLK = 32


def _rnn_body(idx_ref, emb_ref, wih_ref, whh_ref, bih_ref, bhh_ref,
              wout_ref, bout_ref, out_ref, xw_ref, h_ref, acc_ref,
              xbuf_ref, gsem, *, ns, s_total, b_tot):
    i = pl.program_id(0)
    j = pl.program_id(1)
    bh = h_ref.shape[0]
    n_rows = S_BLK * bh

    jb = jnp.minimum(j, ns - 1)
    base = jb * (S_BLK * b_tot) + i * bh
    slot_g = jax.lax.rem(j, 2)

    @pl.when(j < ns)
    def _():
        for t in range(S_BLK):
            for b in range(bh):
                tok = idx_ref[base + t * b_tot + b]
                pltpu.make_async_copy(
                    emb_ref.at[tok],
                    xbuf_ref.at[slot_g, t * bh + b],
                    gsem.at[slot_g],
                ).start()

    slot_c = jax.lax.rem(j + 1, 2)

    @pl.when(j >= 1)
    def _():
        for k in range(n_rows):
            pltpu.make_async_copy(
                emb_ref.at[0],
                xbuf_ref.at[slot_c, k],
                gsem.at[slot_c],
            ).wait()

    xw_ref[...] = (
        jnp.dot(xbuf_ref[slot_c], wih_ref[...],
                preferred_element_type=jnp.float32)
        + bih_ref[...]
    )

    h = h_ref[...]
    acc = acc_ref[...]
    whh = whh_ref[...]
    bhh = bhh_ref[...]
    for t in range(S_BLK):
        xw_t = xw_ref[t * bh:(t + 1) * bh, :]
        h = jnp.tanh(xw_t + jnp.dot(h, whh, preferred_element_type=jnp.float32)
                     + bhh)
        acc = acc + h
    h_ref[...] = h
    acc_ref[...] = acc

    @pl.when(j == 0)
    def _():
        h_ref[...] = jnp.zeros_like(h_ref)
        acc_ref[...] = jnp.zeros_like(acc_ref)

    @pl.when(j == ns)
    def _():
        z = (jnp.dot(acc, wout_ref[...], preferred_element_type=jnp.float32)
             + s_total * bout_ref[...])
        m = jnp.max(z, axis=1, keepdims=True)
        lse = jnp.log(jnp.sum(jnp.exp(z - m), axis=1, keepdims=True)) + m
        out_ref[...] = z - lse


def kernel(inputs, emb, W_ih, W_hh, b_ih, b_hh, W_out, b_out):
    S, B = inputs.shape
    V, E = emb.shape
    H = W_hh.shape[0]
    C = W_out.shape[0]
    ns = S // S_BLK
    bh = B // 2

    idx = inputs.reshape(-1).astype(jnp.int32)

    body = functools.partial(_rnn_body, ns=ns, s_total=float(S), b_tot=B)

    out = pl.pallas_call(
        body,
        out_shape=jax.ShapeDtypeStruct((B, C), jnp.float32),
        grid=(2, ns + 1),
        in_specs=[
            pl.BlockSpec(memory_space=pltpu.SMEM),
            pl.BlockSpec(memory_space=pl.ANY),
            pl.BlockSpec((E, H), lambda i, j: (0, 0)),
            pl.BlockSpec((H, H), lambda i, j: (0, 0)),
            pl.BlockSpec((1, H), lambda i, j: (0, 0)),
            pl.BlockSpec((1, H), lambda i, j: (0, 0)),
            pl.BlockSpec((H, C), lambda i, j: (0, 0)),
            pl.BlockSpec((1, C), lambda i, j: (0, 0)),
        ],
        out_specs=pl.BlockSpec((bh, C), lambda i, j: (i, 0)),
        scratch_shapes=[
            pltpu.VMEM((S_BLK * bh, H), jnp.float32),
            pltpu.VMEM((bh, H), jnp.float32),
            pltpu.VMEM((bh, H), jnp.float32),
            pltpu.VMEM((2, S_BLK * bh, E), jnp.float32),
            pltpu.SemaphoreType.DMA((2,)),
        ],
        compiler_params=pltpu.CompilerParams(
            dimension_semantics=("parallel", "arbitrary"),
        ),
        name="rnn_fused_gather",
    )(
        idx,
        emb,
        W_ih.T,
        W_hh.T,
        b_ih.reshape(1, H),
        b_hh.reshape(1, H),
        W_out.T,
        b_out.reshape(1, C),
    )
    return out


# speedup vs baseline: 4.6654x; 1.0746x over previous
import functools

import jax
import jax.numpy as jnp
from jax.experimental import pallas as pl
from jax.experimental.pallas import tpu as pltpu

S_BLK = 32


def _rnn_body(idx_ref, emb_ref, wih_ref, whh_ref, bih_ref, bhh_ref,
              wout_ref, bout_ref, out_ref, xw_ref, h_ref, acc_ref,
              xbuf_ref, gsem, *, ns, s_total, b_tot):
    i = pl.program_id(0)
    j = pl.program_id(1)
    bh = h_ref.shape[0]
    n_rows = S_BLK * bh

    jb = jnp.minimum(j, ns - 1)
    base = jb * (S_BLK * b_tot) + i * bh
    slot_g = jax.lax.rem(j, 2)
    slot_c = jax.lax.rem(j + 1, 2)

    @pl.when(j >= 1)
    def _():
        for k in range(n_rows):
            pltpu.make_async_copy(
                emb_ref.at[pl.ds(0, 1), :],
                xbuf_ref.at[slot_c, k],
                gsem.at[slot_c],
            ).wait()

    e_dim = emb_ref.shape[1]
    xw_ref[...] = (
        jnp.dot(xbuf_ref[slot_c].reshape(n_rows, e_dim), wih_ref[...],
                preferred_element_type=jnp.float32)
        + bih_ref[...]
    )

    @pl.when(j < ns)
    def _():
        for t in range(S_BLK):
            for b in range(bh):
                tok = idx_ref[base + t * b_tot + b]
                pltpu.make_async_copy(
                    emb_ref.at[pl.ds(tok, 1), :],
                    xbuf_ref.at[slot_g, t * bh + b],
                    gsem.at[slot_g],
                ).start()

    h = h_ref[...]
    acc = acc_ref[...]
    whh = whh_ref[...]
    bhh = bhh_ref[...]
    for t in range(S_BLK):
        xw_t = xw_ref[t * bh:(t + 1) * bh, :]
        h = jnp.tanh(xw_t + jnp.dot(h, whh, preferred_element_type=jnp.float32)
                     + bhh)
        acc = acc + h
    h_ref[...] = h
    acc_ref[...] = acc

    @pl.when(j == 0)
    def _():
        h_ref[...] = jnp.zeros_like(h_ref)
        acc_ref[...] = jnp.zeros_like(acc_ref)

    @pl.when(j == ns)
    def _():
        z = (jnp.dot(acc, wout_ref[...], preferred_element_type=jnp.float32)
             + s_total * bout_ref[...])
        m = jnp.max(z, axis=1, keepdims=True)
        lse = jnp.log(jnp.sum(jnp.exp(z - m), axis=1, keepdims=True)) + m
        out_ref[...] = z - lse


def kernel(inputs, emb, W_ih, W_hh, b_ih, b_hh, W_out, b_out):
    S, B = inputs.shape
    V, E = emb.shape
    H = W_hh.shape[0]
    C = W_out.shape[0]
    ns = S // S_BLK
    bh = B // 2

    idx = inputs.reshape(-1).astype(jnp.int32)

    body = functools.partial(_rnn_body, ns=ns, s_total=float(S), b_tot=B)

    out = pl.pallas_call(
        body,
        out_shape=jax.ShapeDtypeStruct((B, C), jnp.float32),
        grid=(2, ns + 1),
        in_specs=[
            pl.BlockSpec(memory_space=pltpu.SMEM),
            pl.BlockSpec(memory_space=pl.ANY),
            pl.BlockSpec((E, H), lambda i, j: (0, 0)),
            pl.BlockSpec((H, H), lambda i, j: (0, 0)),
            pl.BlockSpec((1, H), lambda i, j: (0, 0)),
            pl.BlockSpec((1, H), lambda i, j: (0, 0)),
            pl.BlockSpec((H, C), lambda i, j: (0, 0)),
            pl.BlockSpec((1, C), lambda i, j: (0, 0)),
        ],
        out_specs=pl.BlockSpec((bh, C), lambda i, j: (i, 0)),
        scratch_shapes=[
            pltpu.VMEM((S_BLK * bh, H), jnp.float32),
            pltpu.VMEM((bh, H), jnp.float32),
            pltpu.VMEM((bh, H), jnp.float32),
            pltpu.VMEM((2, S_BLK * bh, 1, E), jnp.float32),
            pltpu.SemaphoreType.DMA((2,)),
        ],
        compiler_params=pltpu.CompilerParams(
            dimension_semantics=("parallel", "arbitrary"),
        ),
        name="rnn_fused_gather",
    )(
        idx,
        emb,
        W_ih.T,
        W_hh.T,
        b_ih.reshape(1, H),
        b_hh.reshape(1, H),
        W_out.T,
        b_out.reshape(1, C),
    )
    return out
